# baseline (speedup 1.0000x reference)
import functools

import jax
import jax.numpy as jnp
from jax import lax
from jax.experimental import pallas as pl
from jax.experimental.pallas import tpu as pltpu
from jax.experimental.pallas import tpu_sc as plsc

_B, _Mb, _Nb, _C = 4, 8192, 4096, 128
_N = _B * _Nb

_QBLK = 128
_SLC = 128

_INF = 3.0e38
_BIGI = 2**30


def _nn3_kernel(q_ref, xt_ref, w_ref, idx_ref):
    b = pl.program_id(0)
    qx = jnp.broadcast_to(q_ref[0, :, 0:1], (_QBLK, _SLC))
    qy = jnp.broadcast_to(q_ref[0, :, 1:2], (_QBLK, _SLC))
    qz = jnp.broadcast_to(q_ref[0, :, 2:3], (_QBLK, _SLC))
    lane = lax.broadcasted_iota(jnp.int32, (1, _SLC), 1)

    def slice_step(c, st):
        b1, b2, b3, i1, i2, i3 = st
        off = c * _SLC
        xsx = xt_ref[0, 0:1, pl.ds(off, _SLC)]
        xsy = xt_ref[0, 1:2, pl.ds(off, _SLC)]
        xsz = xt_ref[0, 2:3, pl.ds(off, _SLC)]
        dx = qx - xsx
        dy = qy - xsy
        dz = qz - xsz
        v = dx * dx + dy * dy + dz * dz
        cf = c.astype(jnp.float32)
        c1 = v < b1
        c2 = v < b2
        c3 = v < b3
        nb1 = jnp.minimum(v, b1)
        nb2 = jnp.minimum(jnp.maximum(v, b1), b2)
        nb3 = jnp.minimum(jnp.maximum(v, b2), b3)
        ni1 = jnp.where(c1, cf, i1)
        ni2 = jnp.where(c1, i1, jnp.where(c2, cf, i2))
        ni3 = jnp.where(c2, i2, jnp.where(c3, cf, i3))
        return nb1, nb2, nb3, ni1, ni2, ni3

    finit = jnp.full((_QBLK, _SLC), _INF, dtype=jnp.float32)
    b1, b2, b3, i1, i2, i3 = lax.fori_loop(
        0, _Mb // _SLC, slice_step,
        (finit, finit, finit, finit, finit, finit), unroll=2)

    lane_f = lane.astype(jnp.float32)
    g1 = i1 * float(_SLC) + lane_f
    g2 = i2 * float(_SLC) + lane_f
    g3 = i3 * float(_SLC) + lane_f
    off = b * _Mb
    for k in range(3):
        m = jnp.min(b1, axis=1, keepdims=True)
        im = jnp.min(jnp.where(b1 <= m, g1, _INF), axis=1, keepdims=True)
        d = jnp.sqrt(jnp.maximum(m, 0.0))
        w_ref[:, k:k + 1] = 1.0 / (d + 1e-8)
        idx_ref[:, k:k + 1] = im.astype(jnp.int32) + off
        if k < 2:
            cond = g1 == im
            b1 = jnp.where(cond, b2, b1)
            g1 = jnp.where(cond, g2, g1)
            b2 = jnp.where(cond, b3, b2)
            g2 = jnp.where(cond, g3, g2)
            b3 = jnp.where(cond, _INF, b3)
    w = w_ref[...]
    w_ref[...] = w / jnp.sum(w, axis=1, keepdims=True)


def _three_nn_weights(new_xyz, xyz):
    qb = new_xyz.reshape(_B, _Nb, 3)
    xt = xyz.reshape(_B, _Mb, 3).transpose(0, 2, 1)
    nq = _Nb // _QBLK
    w_q, idx_q = pl.pallas_call(
        _nn3_kernel,
        grid=(_B, nq),
        in_specs=[
            pl.BlockSpec((1, _QBLK, 3), lambda b, i: (b, i, 0)),
            pl.BlockSpec((1, 3, _Mb), lambda b, i: (b, 0, 0)),
        ],
        out_specs=[
            pl.BlockSpec((_QBLK, 3), lambda b, i: (b * nq + i, 0)),
            pl.BlockSpec((_QBLK, 3), lambda b, i: (b * nq + i, 0)),
        ],
        out_shape=[
            jax.ShapeDtypeStruct((_N, 3), jnp.float32),
            jax.ShapeDtypeStruct((_N, 3), jnp.int32),
        ],
    )(qb, xt)
    return w_q.T, idx_q.T


_NW = 32
_NQW = _N // _NW
_CH = 128


def _interp_sc(features, w_t, idx_t):
    mesh = plsc.VectorSubcoreMesh(core_axis_name="c", subcore_axis_name="s")

    @functools.partial(
        pl.kernel,
        mesh=mesh,
        out_type=jax.ShapeDtypeStruct((_N, _C), jnp.float32),
        scratch_types=[
            pltpu.VMEM((_CH,), jnp.int32),
            pltpu.VMEM((_CH,), jnp.int32),
            pltpu.VMEM((_CH,), jnp.int32),
            pltpu.VMEM((_CH, _C), jnp.float32),
            pltpu.VMEM((_CH, _C), jnp.float32),
            pltpu.VMEM((_CH, _C), jnp.float32),
            pltpu.VMEM((3, _CH), jnp.float32),
            pltpu.VMEM((_CH, _C), jnp.float32),
            pltpu.SemaphoreType.DMA,
        ],
    )
    def k(feat_hbm, w_hbm, i1_hbm, i2_hbm, i3_hbm, out_hbm,
          i1_v, i2_v, i3_v, r1_v, r2_v, r3_v, w_v, out_v, sem):
        wid = lax.axis_index("s") * 2 + lax.axis_index("c")

        def do_chunk(ch, _):
            base = wid * _NQW + ch * _CH
            pltpu.sync_copy(i1_hbm.at[pl.ds(base, _CH)], i1_v)
            pltpu.sync_copy(i2_hbm.at[pl.ds(base, _CH)], i2_v)
            pltpu.sync_copy(i3_hbm.at[pl.ds(base, _CH)], i3_v)
            pltpu.sync_copy(w_hbm.at[:, pl.ds(base, _CH)], w_v)
            c1 = pltpu.async_copy(feat_hbm.at[i1_v], r1_v, sem)
            c2 = pltpu.async_copy(feat_hbm.at[i2_v], r2_v, sem)
            c3 = pltpu.async_copy(feat_hbm.at[i3_v], r3_v, sem)
            c1.wait()
            c2.wait()
            c3.wait()

            def per_grp(g2, _):
                w1g = w_v[0, pl.ds(g2 * 16, 16)]
                w2g = w_v[1, pl.ds(g2 * 16, 16)]
                w3g = w_v[2, pl.ds(g2 * 16, 16)]
                for j in range(16):
                    q = g2 * 16 + j
                    w1 = w1g[j]
                    w2 = w2g[j]
                    w3 = w3g[j]
                    for g in range(_C // 16):
                        sl = pl.ds(g * 16, 16)
                        out_v[q, sl] = (w1 * r1_v[q, sl] + w2 * r2_v[q, sl]
                                        + w3 * r3_v[q, sl])
                return 0

            lax.fori_loop(0, _CH // 16, per_grp, 0)
            pltpu.sync_copy(out_v, out_hbm.at[pl.ds(base, _CH)])
            return 0

        lax.fori_loop(0, _NQW // _CH, do_chunk, 0)

    return k(features, w_t, idx_t[0], idx_t[1], idx_t[2])


def kernel(xyz, xyz_batch_cnt, new_xyz, new_xyz_batch_cnt, features):
    w_t, idx_t = _three_nn_weights(new_xyz, xyz)
    return _interp_sc(features, w_t, idx_t)


# speedup vs baseline: 1.2912x; 1.2912x over previous
import functools

import jax
import jax.numpy as jnp
from jax import lax
from jax.experimental import pallas as pl
from jax.experimental.pallas import tpu as pltpu
from jax.experimental.pallas import tpu_sc as plsc

_B, _Mb, _Nb, _C = 4, 8192, 4096, 128
_N = _B * _Nb

_QBLK = 64
_SLC = 128
_GRP = 8

_INF = 3.0e38
_BIGI = 2**30


def _nn3_kernel(q_ref, xt_ref, w_ref, idx_ref):
    b = pl.program_id(0)
    qx = jnp.broadcast_to(q_ref[0, :, 0:1], (_QBLK, _SLC))
    qy = jnp.broadcast_to(q_ref[0, :, 1:2], (_QBLK, _SLC))
    qz = jnp.broadcast_to(q_ref[0, :, 2:3], (_QBLK, _SLC))
    lane = lax.broadcasted_iota(jnp.int32, (1, _SLC), 1)

    def _merge22(a, b):
        a1, a2, ia1, ia2 = a
        b1_, b2_, ib1, ib2 = b
        c1 = a1 <= b1_
        r1 = jnp.minimum(a1, b1_)
        t1 = jnp.maximum(a1, b1_)
        ir1 = jnp.where(c1, ia1, ib1)
        it1 = jnp.where(c1, ib1, ia1)
        c2 = a2 <= b2_
        t2 = jnp.minimum(a2, b2_)
        it2 = jnp.where(c2, ia2, ib2)
        c3 = t1 <= t2
        r2 = jnp.minimum(t1, t2)
        r3 = jnp.maximum(t1, t2)
        ir2 = jnp.where(c3, it1, it2)
        ir3 = jnp.where(c3, it2, it1)
        return r1, r2, r3, ir1, ir2, ir3

    def _merge33(a, b):
        a1, a2, a3, ia1, ia2, ia3 = a
        b1_, b2_, b3_, ib1, ib2, ib3 = b
        c1 = a1 <= b1_
        r1 = jnp.minimum(a1, b1_)
        t1 = jnp.maximum(a1, b1_)
        ir1 = jnp.where(c1, ia1, ib1)
        it1 = jnp.where(c1, ib1, ia1)
        c2 = a2 <= b2_
        t2 = jnp.minimum(a2, b2_)
        it2 = jnp.where(c2, ia2, ib2)
        c3 = t1 <= t2
        r2 = jnp.minimum(t1, t2)
        u = jnp.maximum(t1, t2)
        ir2 = jnp.where(c3, it1, it2)
        iu = jnp.where(c3, it2, it1)
        c4 = a3 <= b3_
        t3 = jnp.minimum(a3, b3_)
        it3 = jnp.where(c4, ia3, ib3)
        c5 = u <= t3
        r3 = jnp.minimum(u, t3)
        ir3 = jnp.where(c5, iu, it3)
        return r1, r2, r3, ir1, ir2, ir3

    def group_step(g, st):
        base = g * _GRP
        vs = []
        for j in range(_GRP):
            off = (base + j) * _SLC
            xsx = xt_ref[0, 0:1, pl.ds(off, _SLC)]
            xsy = xt_ref[0, 1:2, pl.ds(off, _SLC)]
            xsz = xt_ref[0, 2:3, pl.ds(off, _SLC)]
            dx = qx - xsx
            dy = qy - xsy
            dz = qz - xsz
            v = dx * dx + dy * dy + dz * dz
            vs.append((v, (base + j).astype(jnp.float32)))
        pairs = []
        for j in range(0, _GRP, 2):
            va, ia = vs[j]
            vb, ib = vs[j + 1]
            c = va <= vb
            pairs.append((jnp.minimum(va, vb), jnp.maximum(va, vb),
                          jnp.where(c, ia, ib), jnp.where(c, ib, ia)))
        ta = _merge22(pairs[0], pairs[1])
        tb = _merge22(pairs[2], pairs[3])
        grp = _merge33(ta, tb)
        return _merge33(st, grp)

    finit = jnp.full((_QBLK, _SLC), _INF, dtype=jnp.float32)
    b1, b2, b3, i1, i2, i3 = lax.fori_loop(
        0, _Mb // (_SLC * _GRP), group_step,
        (finit, finit, finit, finit, finit, finit), unroll=1)

    lane_f = lane.astype(jnp.float32)
    g1 = i1 * float(_SLC) + lane_f
    g2 = i2 * float(_SLC) + lane_f
    g3 = i3 * float(_SLC) + lane_f
    off = b * _Mb
    for k in range(3):
        m = jnp.min(b1, axis=1, keepdims=True)
        im = jnp.min(jnp.where(b1 <= m, g1, _INF), axis=1, keepdims=True)
        d = jnp.sqrt(jnp.maximum(m, 0.0))
        w_ref[:, k:k + 1] = 1.0 / (d + 1e-8)
        idx_ref[:, k:k + 1] = im.astype(jnp.int32) + off
        if k < 2:
            cond = g1 == im
            b1 = jnp.where(cond, b2, b1)
            g1 = jnp.where(cond, g2, g1)
            b2 = jnp.where(cond, b3, b2)
            g2 = jnp.where(cond, g3, g2)
            b3 = jnp.where(cond, _INF, b3)
    w = w_ref[...]
    w_ref[...] = w / jnp.sum(w, axis=1, keepdims=True)


def _three_nn_weights(new_xyz, xyz):
    qb = new_xyz.reshape(_B, _Nb, 3)
    xt = xyz.reshape(_B, _Mb, 3).transpose(0, 2, 1)
    nq = _Nb // _QBLK
    w_q, idx_q = pl.pallas_call(
        _nn3_kernel,
        grid=(_B, nq),
        in_specs=[
            pl.BlockSpec((1, _QBLK, 3), lambda b, i: (b, i, 0)),
            pl.BlockSpec((1, 3, _Mb), lambda b, i: (b, 0, 0)),
        ],
        out_specs=[
            pl.BlockSpec((_QBLK, 3), lambda b, i: (b * nq + i, 0)),
            pl.BlockSpec((_QBLK, 3), lambda b, i: (b * nq + i, 0)),
        ],
        out_shape=[
            jax.ShapeDtypeStruct((_N, 3), jnp.float32),
            jax.ShapeDtypeStruct((_N, 3), jnp.int32),
        ],
    )(qb, xt)
    return w_q.T, idx_q.T


_NW = 32
_NQW = _N // _NW
_CH = 128


def _interp_sc(features, w_t, idx_t):
    mesh = plsc.VectorSubcoreMesh(core_axis_name="c", subcore_axis_name="s")

    @functools.partial(
        pl.kernel,
        mesh=mesh,
        out_type=jax.ShapeDtypeStruct((_N, _C), jnp.float32),
        scratch_types=[
            pltpu.VMEM((_CH,), jnp.int32),
            pltpu.VMEM((_CH,), jnp.int32),
            pltpu.VMEM((_CH,), jnp.int32),
            pltpu.VMEM((_CH, _C), jnp.float32),
            pltpu.VMEM((_CH, _C), jnp.float32),
            pltpu.VMEM((_CH, _C), jnp.float32),
            pltpu.VMEM((3, _CH), jnp.float32),
            pltpu.VMEM((_CH, _C), jnp.float32),
            pltpu.SemaphoreType.DMA,
        ],
    )
    def k(feat_hbm, w_hbm, i1_hbm, i2_hbm, i3_hbm, out_hbm,
          i1_v, i2_v, i3_v, r1_v, r2_v, r3_v, w_v, out_v, sem):
        wid = lax.axis_index("s") * 2 + lax.axis_index("c")

        def do_chunk(ch, _):
            base = wid * _NQW + ch * _CH
            pltpu.sync_copy(i1_hbm.at[pl.ds(base, _CH)], i1_v)
            pltpu.sync_copy(i2_hbm.at[pl.ds(base, _CH)], i2_v)
            pltpu.sync_copy(i3_hbm.at[pl.ds(base, _CH)], i3_v)
            pltpu.sync_copy(w_hbm.at[:, pl.ds(base, _CH)], w_v)
            c1 = pltpu.async_copy(feat_hbm.at[i1_v], r1_v, sem)
            c2 = pltpu.async_copy(feat_hbm.at[i2_v], r2_v, sem)
            c3 = pltpu.async_copy(feat_hbm.at[i3_v], r3_v, sem)
            c1.wait()
            c2.wait()
            c3.wait()

            def per_grp(g2, _):
                w1g = w_v[0, pl.ds(g2 * 16, 16)]
                w2g = w_v[1, pl.ds(g2 * 16, 16)]
                w3g = w_v[2, pl.ds(g2 * 16, 16)]
                for j in range(16):
                    q = g2 * 16 + j
                    w1 = w1g[j]
                    w2 = w2g[j]
                    w3 = w3g[j]
                    for g in range(_C // 16):
                        sl = pl.ds(g * 16, 16)
                        out_v[q, sl] = (w1 * r1_v[q, sl] + w2 * r2_v[q, sl]
                                        + w3 * r3_v[q, sl])
                return 0

            lax.fori_loop(0, _CH // 16, per_grp, 0)
            pltpu.sync_copy(out_v, out_hbm.at[pl.ds(base, _CH)])
            return 0

        lax.fori_loop(0, _NQW // _CH, do_chunk, 0)

    return k(features, w_t, idx_t[0], idx_t[1], idx_t[2])


def kernel(xyz, xyz_batch_cnt, new_xyz, new_xyz_batch_cnt, features):
    w_t, idx_t = _three_nn_weights(new_xyz, xyz)
    return _interp_sc(features, w_t, idx_t)


# speedup vs baseline: 1.8566x; 1.4379x over previous
import functools

import jax
import jax.numpy as jnp
from jax import lax
from jax.experimental import pallas as pl
from jax.experimental.pallas import tpu as pltpu
from jax.experimental.pallas import tpu_sc as plsc

_B, _Mb, _Nb, _C = 4, 8192, 4096, 128
_N = _B * _Nb

_QBLK = 64
_SLC = 128
_GRP = 16

_INF = 3.0e38
_BIGI = 2**30


def _nn3_kernel(q_ref, xt_ref, w_ref, idx_ref):
    b = pl.program_id(0)
    qx = jnp.broadcast_to(q_ref[0, :, 0:1], (_QBLK, _SLC))
    qy = jnp.broadcast_to(q_ref[0, :, 1:2], (_QBLK, _SLC))
    qz = jnp.broadcast_to(q_ref[0, :, 2:3], (_QBLK, _SLC))
    lane = lax.broadcasted_iota(jnp.int32, (1, _SLC), 1)

    def _merge22(a, b):
        a1, a2, ia1, ia2 = a
        b1_, b2_, ib1, ib2 = b
        c1 = a1 <= b1_
        r1 = jnp.minimum(a1, b1_)
        t1 = jnp.maximum(a1, b1_)
        ir1 = jnp.where(c1, ia1, ib1)
        it1 = jnp.where(c1, ib1, ia1)
        c2 = a2 <= b2_
        t2 = jnp.minimum(a2, b2_)
        it2 = jnp.where(c2, ia2, ib2)
        c3 = t1 <= t2
        r2 = jnp.minimum(t1, t2)
        r3 = jnp.maximum(t1, t2)
        ir2 = jnp.where(c3, it1, it2)
        ir3 = jnp.where(c3, it2, it1)
        return r1, r2, r3, ir1, ir2, ir3

    def _merge33(a, b):
        a1, a2, a3, ia1, ia2, ia3 = a
        b1_, b2_, b3_, ib1, ib2, ib3 = b
        c1 = a1 <= b1_
        r1 = jnp.minimum(a1, b1_)
        t1 = jnp.maximum(a1, b1_)
        ir1 = jnp.where(c1, ia1, ib1)
        it1 = jnp.where(c1, ib1, ia1)
        c2 = a2 <= b2_
        t2 = jnp.minimum(a2, b2_)
        it2 = jnp.where(c2, ia2, ib2)
        c3 = t1 <= t2
        r2 = jnp.minimum(t1, t2)
        u = jnp.maximum(t1, t2)
        ir2 = jnp.where(c3, it1, it2)
        iu = jnp.where(c3, it2, it1)
        c4 = a3 <= b3_
        t3 = jnp.minimum(a3, b3_)
        it3 = jnp.where(c4, ia3, ib3)
        c5 = u <= t3
        r3 = jnp.minimum(u, t3)
        ir3 = jnp.where(c5, iu, it3)
        return r1, r2, r3, ir1, ir2, ir3

    def group_step(g, st):
        base = g * _GRP
        vs = []
        for j in range(_GRP):
            off = (base + j) * _SLC
            xsx = xt_ref[0, 0:1, pl.ds(off, _SLC)]
            xsy = xt_ref[0, 1:2, pl.ds(off, _SLC)]
            xsz = xt_ref[0, 2:3, pl.ds(off, _SLC)]
            dx = qx - xsx
            dy = qy - xsy
            dz = qz - xsz
            v = dx * dx + dy * dy + dz * dz
            vs.append((v, (base + j).astype(jnp.float32)))
        pairs = []
        for j in range(0, _GRP, 2):
            va, ia = vs[j]
            vb, ib = vs[j + 1]
            c = va <= vb
            pairs.append((jnp.minimum(va, vb), jnp.maximum(va, vb),
                          jnp.where(c, ia, ib), jnp.where(c, ib, ia)))
        ta = _merge22(pairs[0], pairs[1])
        tb = _merge22(pairs[2], pairs[3])
        grp = _merge33(ta, tb)
        return _merge33(st, grp)

    finit = jnp.full((_QBLK, _SLC), _INF, dtype=jnp.float32)
    b1, b2, b3, i1, i2, i3 = lax.fori_loop(
        0, _Mb // (_SLC * _GRP), group_step,
        (finit, finit, finit, finit, finit, finit), unroll=1)

    lane_f = lane.astype(jnp.float32)
    g1 = i1 * float(_SLC) + lane_f
    g2 = i2 * float(_SLC) + lane_f
    g3 = i3 * float(_SLC) + lane_f
    off = b * _Mb
    for k in range(3):
        m = jnp.min(b1, axis=1, keepdims=True)
        im = jnp.min(jnp.where(b1 <= m, g1, _INF), axis=1, keepdims=True)
        d = jnp.sqrt(jnp.maximum(m, 0.0))
        w_ref[:, k:k + 1] = 1.0 / (d + 1e-8)
        idx_ref[:, k:k + 1] = im.astype(jnp.int32) + off
        if k < 2:
            cond = g1 == im
            b1 = jnp.where(cond, b2, b1)
            g1 = jnp.where(cond, g2, g1)
            b2 = jnp.where(cond, b3, b2)
            g2 = jnp.where(cond, g3, g2)
            b3 = jnp.where(cond, _INF, b3)
    w = w_ref[...]
    w_ref[...] = w / jnp.sum(w, axis=1, keepdims=True)


def _three_nn_weights(new_xyz, xyz):
    qb = new_xyz.reshape(_B, _Nb, 3)
    xt = xyz.reshape(_B, _Mb, 3).transpose(0, 2, 1)
    nq = _Nb // _QBLK
    w_q, idx_q = pl.pallas_call(
        _nn3_kernel,
        grid=(_B, nq),
        in_specs=[
            pl.BlockSpec((1, _QBLK, 3), lambda b, i: (b, i, 0)),
            pl.BlockSpec((1, 3, _Mb), lambda b, i: (b, 0, 0)),
        ],
        out_specs=[
            pl.BlockSpec((_QBLK, 3), lambda b, i: (b * nq + i, 0)),
            pl.BlockSpec((_QBLK, 3), lambda b, i: (b * nq + i, 0)),
        ],
        out_shape=[
            jax.ShapeDtypeStruct((_N, 3), jnp.float32),
            jax.ShapeDtypeStruct((_N, 3), jnp.int32),
        ],
    )(qb, xt)
    return w_q.T, idx_q.T


_NW = 32
_NQW = _N // _NW
_CH = 128


def _interp_sc(features, w_t, idx_t):
    mesh = plsc.VectorSubcoreMesh(core_axis_name="c", subcore_axis_name="s")

    @functools.partial(
        pl.kernel,
        mesh=mesh,
        out_type=jax.ShapeDtypeStruct((_N, _C), jnp.float32),
        scratch_types=[
            pltpu.VMEM((_CH,), jnp.int32),
            pltpu.VMEM((_CH,), jnp.int32),
            pltpu.VMEM((_CH,), jnp.int32),
            pltpu.VMEM((_CH, _C), jnp.float32),
            pltpu.VMEM((_CH, _C), jnp.float32),
            pltpu.VMEM((_CH, _C), jnp.float32),
            pltpu.VMEM((3, _CH), jnp.float32),
            pltpu.VMEM((_CH, _C), jnp.float32),
            pltpu.SemaphoreType.DMA,
        ],
    )
    def k(feat_hbm, w_hbm, i1_hbm, i2_hbm, i3_hbm, out_hbm,
          i1_v, i2_v, i3_v, r1_v, r2_v, r3_v, w_v, out_v, sem):
        wid = lax.axis_index("s") * 2 + lax.axis_index("c")

        def do_chunk(ch, _):
            base = wid * _NQW + ch * _CH
            pltpu.sync_copy(i1_hbm.at[pl.ds(base, _CH)], i1_v)
            pltpu.sync_copy(i2_hbm.at[pl.ds(base, _CH)], i2_v)
            pltpu.sync_copy(i3_hbm.at[pl.ds(base, _CH)], i3_v)
            pltpu.sync_copy(w_hbm.at[:, pl.ds(base, _CH)], w_v)
            c1 = pltpu.async_copy(feat_hbm.at[i1_v], r1_v, sem)
            c2 = pltpu.async_copy(feat_hbm.at[i2_v], r2_v, sem)
            c3 = pltpu.async_copy(feat_hbm.at[i3_v], r3_v, sem)
            c1.wait()
            c2.wait()
            c3.wait()

            def per_grp(g2, _):
                w1g = w_v[0, pl.ds(g2 * 16, 16)]
                w2g = w_v[1, pl.ds(g2 * 16, 16)]
                w3g = w_v[2, pl.ds(g2 * 16, 16)]
                for j in range(16):
                    q = g2 * 16 + j
                    w1 = w1g[j]
                    w2 = w2g[j]
                    w3 = w3g[j]
                    for g in range(_C // 16):
                        sl = pl.ds(g * 16, 16)
                        out_v[q, sl] = (w1 * r1_v[q, sl] + w2 * r2_v[q, sl]
                                        + w3 * r3_v[q, sl])
                return 0

            lax.fori_loop(0, _CH // 16, per_grp, 0)
            pltpu.sync_copy(out_v, out_hbm.at[pl.ds(base, _CH)])
            return 0

        lax.fori_loop(0, _NQW // _CH, do_chunk, 0)

    return k(features, w_t, idx_t[0], idx_t[1], idx_t[2])


def kernel(xyz, xyz_batch_cnt, new_xyz, new_xyz_batch_cnt, features):
    w_t, idx_t = _three_nn_weights(new_xyz, xyz)
    return _interp_sc(features, w_t, idx_t)
